# Initial kernel scaffold; baseline (speedup 1.0000x reference)
import jax
import jax.numpy as jnp
from jax.experimental import pallas as pl


def kernel(data, emb0, emb1):
    raise NotImplementedError("write your pallas kernel here")


# speedup vs baseline: 1.2921x; 1.2921x over previous
import functools

import jax
import jax.numpy as jnp
from jax import lax
from jax.experimental import pallas as pl
from jax.experimental.pallas import tpu as pltpu
from jax.experimental.pallas import tpu_sc as plsc

_B = 4096
_V = 100000
_D = 64
_W = 5
_NEG = 5
_NW = 32
_BPW = _B // _NW
_NF = 2 * _W + 2 + 2 * _NEG
_NCH = _D // 16


def _sc_body(data_hbm, emb0_hbm, emb1_hbm, out_hbm,
             data_v, acc, rb0, rb1, wb, nb0, nb1, nb2, nb3, nb4,
             pos_v, neg_v,
             s_acc, s_r0, s_r1, s_w, s_n0, s_n1, s_n2, s_n3, s_n4):
    wid = lax.axis_index("s") * 2 + lax.axis_index("c")
    base = wid * _BPW

    pltpu.sync_copy(data_hbm.at[wid], data_v)

    ring = [rb0, rb1]
    ring_sems = [s_r0, s_r1]
    nbufs = [nb0, nb1, nb2, nb3, nb4]
    nsems = [s_n0, s_n1, s_n2, s_n3, s_n4]

    cps = {
        0: pltpu.async_copy(emb0_hbm.at[data_v.at[0]], acc, s_acc),
        1: pltpu.async_copy(emb0_hbm.at[data_v.at[1]], rb0, s_r0),
        2: pltpu.async_copy(emb0_hbm.at[data_v.at[2]], rb1, s_r1),
    }
    cw = pltpu.async_copy(emb1_hbm.at[data_v.at[2 * _W + 1]], wb, s_w)
    cns = [
        pltpu.async_copy(emb1_hbm.at[data_v.at[2 * _W + 2 + n]],
                         nbufs[n], nsems[n])
        for n in range(_NEG)
    ]

    cps[0].wait()
    for j in range(1, 2 * _W):
        slot = (j - 1) % 2
        buf = ring[slot]
        cps[j].wait()

        def add_body(i, _, buf=buf):
            for c in range(_NCH):
                sl = pl.ds(c * 16, 16)
                plsc.addupdate(acc.at[i, sl], buf[i, sl])
            return 0

        lax.fori_loop(0, _BPW, add_body, 0)
        nxt = j + 2
        if nxt < 2 * _W:
            cps[nxt] = pltpu.async_copy(
                emb0_hbm.at[data_v.at[nxt]], buf, ring_sems[slot])

    cw.wait()
    for c in cns:
        c.wait()

    lane = lax.iota(jnp.int32, 16)
    zeros = jnp.zeros((16,), jnp.float32)
    for g in range(_BPW // 16):
        bidx = lane + (g * 16)

        def dot_step(d, carry):
            dvec = jnp.full((16,), d, jnp.int32)
            av = plsc.load_gather(acc, [bidx, dvec])
            pos_p = carry[0] + av * plsc.load_gather(wb, [bidx, dvec])
            negs = tuple(
                carry[1 + n] + av * plsc.load_gather(nbufs[n], [bidx, dvec])
                for n in range(_NEG))
            return (pos_p,) + negs

        res = lax.fori_loop(0, _D, dot_step, (zeros,) * (1 + _NEG))
        pos_v[pl.ds(g * 16, 16)] = res[0]
        for n in range(_NEG):
            neg_v[pl.ds(n * _BPW + g * 16, 16)] = res[1 + n]

    pltpu.sync_copy(pos_v, out_hbm.at[pl.ds(base, _BPW)])
    for n in range(_NEG):
        pltpu.sync_copy(neg_v.at[pl.ds(n * _BPW, _BPW)],
                        out_hbm.at[pl.ds((1 + n) * _B + base, _BPW)])


_sc_kernel = functools.partial(
    pl.kernel,
    out_type=jax.ShapeDtypeStruct(((1 + _NEG) * _B,), jnp.float32),
    mesh=plsc.VectorSubcoreMesh(core_axis_name="c", subcore_axis_name="s"),
    compiler_params=pltpu.CompilerParams(
        needs_layout_passes=False, use_tc_tiling_on_sc=False),
    scratch_types=[
        pltpu.VMEM((_NF, _BPW), jnp.int32),
        *[pltpu.VMEM((_BPW, _D), jnp.float32) for _ in range(9)],
        pltpu.VMEM((_BPW,), jnp.float32),
        pltpu.VMEM((_NEG * _BPW,), jnp.float32),
        *[pltpu.SemaphoreType.DMA for _ in range(9)],
    ],
)(_sc_body)


def _loss_body(d_ref, s_ref, o_ref):
    lens = d_ref[:, 2 * _W, :].astype(jnp.float32)
    x = jnp.clip(s_ref[0] / lens, -10.0, 10.0)
    total = jnp.sum(jnp.log(1.0 + jnp.exp(-x)))
    for n in range(_NEG):
        m = d_ref[:, 2 * _W + 2 + _NEG + n, :].astype(jnp.float32)
        ips = s_ref[1 + n] / lens * m
        z = jnp.clip(-ips, -10.0, 10.0)
        total = total + jnp.sum(jnp.log(1.0 + jnp.exp(-z)) * m)
    o_ref[...] = jnp.reshape(total, (1, 1))


def kernel(data, emb0, emb1):
    data_r = data.T.reshape(_NF, _NW, _BPW).transpose(1, 0, 2)
    raw = _sc_kernel(data_r, emb0, emb1)
    sc6 = raw.reshape(1 + _NEG, _NW, _BPW)
    loss = pl.pallas_call(
        _loss_body,
        out_shape=jax.ShapeDtypeStruct((1, 1), jnp.float32),
    )(data_r, sc6)
    return loss[0, 0]
